# baseline (speedup 1.0000x reference)
import functools

import jax
import jax.numpy as jnp
from jax import lax
from jax.experimental import pallas as pl
from jax.experimental.pallas import tpu as pltpu
from jax.experimental.pallas import tpu_sc as plsc

NC, NS, L = 2, 16, 16
NW = NC * NS
IW = 128
G = 2
CHUNK = G * IW
NBUF = 3


def _build_sc_call(N, V, D, VROWS):
    n_per_w = N // NW
    n_chunks = n_per_w // CHUNK
    rows_per_tile = VROWS // NS
    mesh = plsc.VectorSubcoreMesh(
        core_axis_name="c", subcore_axis_name="s",
        num_cores=NC, num_subcores=NS)

    @functools.partial(
        pl.kernel,
        out_type=jax.ShapeDtypeStruct((N, D), jnp.float32),
        mesh=mesh,
        scratch_types=[
            pltpu.VMEM_SHARED((VROWS, D), jnp.float32),
            pltpu.VMEM((8, D), jnp.float32),
            pltpu.VMEM((D,), jnp.float32),
            [pltpu.VMEM((G, IW), jnp.int32) for _ in range(NBUF)],
            [pltpu.VMEM((G, IW), jnp.int32) for _ in range(NBUF)],
            [pltpu.VMEM((G, IW), jnp.int32) for _ in range(NBUF)],
            [pltpu.VMEM((CHUNK, D), jnp.float32) for _ in range(NBUF)],
            [pltpu.SemaphoreType.DMA for _ in range(NBUF)],
            [pltpu.SemaphoreType.DMA for _ in range(NBUF)],
            [pltpu.SemaphoreType.DMA for _ in range(NBUF)],
        ],
    )
    def sc_fn(ids_hbm, msk_hbm, table_hbm, bias_hbm, out_hbm,
              aug_sh, row1_v, bias_v, in_v, msk_v, eff_v, row_v, sem_i,
              sem_o, sem_g):
        cid = lax.axis_index("c")
        sid = lax.axis_index("s")
        wid = sid * NC + cid

        gbase = wid * (n_per_w // IW)

        def fire_input(c, b):
            pltpu.async_copy(ids_hbm.at[pl.ds(gbase + c * G, G)],
                             in_v[b], sem_i[b])
            pltpu.async_copy(msk_hbm.at[pl.ds(gbase + c * G, G)],
                             msk_v[b], sem_i[b])

        def wait_input(c, b):
            pltpu.make_async_copy(ids_hbm.at[pl.ds(gbase + c * G, G)],
                                  in_v[b], sem_i[b]).wait()
            pltpu.make_async_copy(msk_hbm.at[pl.ds(gbase + c * G, G)],
                                  msk_v[b], sem_i[b]).wait()

        def wait_output(b):
            pltpu.make_async_copy(row_v[b], out_hbm.at[pl.ds(0, CHUNK)],
                                  sem_o[b]).wait()

        for b in range(NBUF):
            fire_input(b, b)

        NTAB = VROWS // 8
        base = pl.multiple_of(sid * 8, 8)

        @pl.when(sid < NTAB)
        def _table_rows():
            bias_d = pltpu.async_copy(bias_hbm, bias_v, sem_g[0])
            rows_d = pltpu.async_copy(table_hbm.at[pl.ds(base, 8)], row1_v,
                                      sem_g[1])
            bias_d.wait()
            rows_d.wait()
            for k in range(8):
                @pl.when(base + k < V)
                def _add():
                    for j in range(D // L):
                        sl = pl.ds(j * L, L)
                        row1_v[k, sl] = row1_v[k, sl] + bias_v[sl]
            pltpu.sync_copy(row1_v, aug_sh.at[pl.ds(base, 8)])

        plsc.subcore_barrier()

        def fire_gather(b):
            for g in range(G):
                pltpu.async_copy(aug_sh.at[eff_v[b].at[g]],
                                 row_v[b].at[pl.ds(g * IW, IW)],
                                 sem_g[b])

        def wait_gather(b):
            for g in range(G):
                pltpu.make_async_copy(aug_sh.at[eff_v[b].at[g]],
                                      row_v[b].at[pl.ds(g * IW, IW)],
                                      sem_g[b]).wait()

        def stage_front(c, b, fire_in, wait_out):
            wait_input(c, b)

            def sel(k, c2):
                sl = pl.ds(k * L, L)
                for g in range(G):
                    idv = in_v[b][g, sl]
                    mv = msk_v[b][g, sl]
                    eff_v[b][g, sl] = jnp.where(mv != 0, V, idv)
                return c2

            lax.fori_loop(0, IW // L, sel, 0)

            if isinstance(fire_in, bool):
                if fire_in:
                    fire_input(c + NBUF, b)
            else:
                @pl.when(fire_in)
                def _():
                    fire_input(c + NBUF, b)

            if wait_out:
                wait_output(b)
            fire_gather(b)

        def stage_back(c, b):
            wait_gather(b)
            pltpu.async_copy(row_v[b],
                             out_hbm.at[pl.ds((gbase + c * G) * IW, CHUNK)],
                             sem_o[b])

        stage_front(0, 0, fire_in=bool(NBUF < n_chunks), wait_out=False)
        for c in range(0, NBUF - 1):
            stage_back(c, c % NBUF)
            stage_front(c + 1, (c + 1) % NBUF,
                        fire_in=bool(c + 1 + NBUF < n_chunks),
                        wait_out=False)
        n_main = ((n_chunks - 1) // NBUF) * NBUF

        for c in range(NBUF - 1, NBUF):
            stage_back(c, c % NBUF)
            stage_front(c + 1, (c + 1) % NBUF,
                        fire_in=bool(c + 1 + NBUF < n_chunks),
                        wait_out=True)

        def outer(o, carry):
            for b in range(NBUF):
                c = o * NBUF + b
                stage_back(c, b)
                stage_front(c + 1, (b + 1) % NBUF,
                            fire_in=(c + 1 + NBUF < n_chunks),
                            wait_out=True)
            return carry

        lax.fori_loop(1, n_main // NBUF, outer, 0)
        for c in range(n_main, n_chunks - 1):
            stage_back(c, c % NBUF)
            stage_front(c + 1, (c + 1) % NBUF, fire_in=False, wait_out=True)
        stage_back(n_chunks - 1, (n_chunks - 1) % NBUF)

        for b in range(NBUF):
            wait_output(b)

    return sc_fn


def kernel(phoneme_ids, padding_mask, table, pos_bias):
    B, T = phoneme_ids.shape
    V, D = table.shape
    N = B * T
    VROWS = ((V + 1 + NS - 1) // NS) * NS

    ids = phoneme_ids.reshape(N // IW, IW).astype(jnp.int32)
    mask = padding_mask.reshape(N // IW, IW).astype(jnp.int32)
    bias = pos_bias.reshape(D).astype(jnp.float32)
    table_p = jnp.concatenate(
        [table.astype(jnp.float32), jnp.zeros((VROWS - V, D), jnp.float32)],
        axis=0)

    sc_fn = _build_sc_call(N, V, D, VROWS)
    out = sc_fn(ids, mask, table_p, bias)
    return out.reshape(B, T, D)


# speedup vs baseline: 1.0114x; 1.0114x over previous
import functools

import jax
import jax.numpy as jnp
from jax import lax
from jax.experimental import pallas as pl
from jax.experimental.pallas import tpu as pltpu
from jax.experimental.pallas import tpu_sc as plsc

NC, NS, L = 2, 16, 16
NW = NC * NS
IW = 128
G = 1
CHUNK = G * IW
NBUF = 4


def _build_sc_call(N, V, D, VROWS):
    n_per_w = N // NW
    n_chunks = n_per_w // CHUNK
    rows_per_tile = VROWS // NS
    mesh = plsc.VectorSubcoreMesh(
        core_axis_name="c", subcore_axis_name="s",
        num_cores=NC, num_subcores=NS)

    @functools.partial(
        pl.kernel,
        out_type=jax.ShapeDtypeStruct((N, D), jnp.float32),
        mesh=mesh,
        scratch_types=[
            pltpu.VMEM_SHARED((VROWS, D), jnp.float32),
            pltpu.VMEM((8, D), jnp.float32),
            pltpu.VMEM((D,), jnp.float32),
            [pltpu.VMEM((G, 2 * IW), jnp.int32) for _ in range(NBUF)],
            [pltpu.VMEM((G, IW), jnp.int32) for _ in range(NBUF)],
            [pltpu.VMEM((CHUNK, D), jnp.float32) for _ in range(NBUF)],
            [pltpu.SemaphoreType.DMA for _ in range(NBUF)],
            [pltpu.SemaphoreType.DMA for _ in range(NBUF)],
            [pltpu.SemaphoreType.DMA for _ in range(NBUF)],
        ],
    )
    def sc_fn(idm_hbm, table_hbm, bias_hbm, out_hbm,
              aug_sh, row1_v, bias_v, in_v, eff_v, row_v, sem_i, sem_o,
              sem_g):
        cid = lax.axis_index("c")
        sid = lax.axis_index("s")
        wid = sid * NC + cid

        gbase = wid * (n_per_w // IW)

        def fire_input(c, b):
            pltpu.async_copy(idm_hbm.at[pl.ds(gbase + c * G, G)],
                             in_v[b], sem_i[b])

        def wait_input(c, b):
            pltpu.make_async_copy(idm_hbm.at[pl.ds(gbase + c * G, G)],
                                  in_v[b], sem_i[b]).wait()

        def wait_output(b):
            pltpu.make_async_copy(row_v[b], out_hbm.at[pl.ds(0, CHUNK)],
                                  sem_o[b]).wait()

        for b in range(NBUF):
            fire_input(b, b)

        NTAB = VROWS // 8
        base = pl.multiple_of(sid * 8, 8)

        @pl.when(sid < NTAB)
        def _table_rows():
            bias_d = pltpu.async_copy(bias_hbm, bias_v, sem_g[0])
            rows_d = pltpu.async_copy(table_hbm.at[pl.ds(base, 8)], row1_v,
                                      sem_g[1])
            bias_d.wait()
            rows_d.wait()
            for k in range(8):
                @pl.when(base + k < V)
                def _add():
                    for j in range(D // L):
                        sl = pl.ds(j * L, L)
                        row1_v[k, sl] = row1_v[k, sl] + bias_v[sl]
            pltpu.sync_copy(row1_v, aug_sh.at[pl.ds(base, 8)])

        plsc.subcore_barrier()

        def fire_gather(b):
            for g in range(G):
                pltpu.async_copy(aug_sh.at[eff_v[b].at[g]],
                                 row_v[b].at[pl.ds(g * IW, IW)],
                                 sem_g[b])

        def wait_gather(b):
            for g in range(G):
                pltpu.make_async_copy(aug_sh.at[eff_v[b].at[g]],
                                      row_v[b].at[pl.ds(g * IW, IW)],
                                      sem_g[b]).wait()

        def stage_front(c, b, fire_in, wait_out):
            wait_input(c, b)

            def sel(k, c2):
                sl = pl.ds(k * L, L)
                msl = pl.ds(k * L + IW, L)
                for g in range(G):
                    idv = in_v[b][g, sl]
                    mv = in_v[b][g, msl]
                    eff_v[b][g, sl] = jnp.where(mv != 0, V, idv)
                return c2

            lax.fori_loop(0, IW // L, sel, 0)

            if isinstance(fire_in, bool):
                if fire_in:
                    fire_input(c + NBUF, b)
            else:
                @pl.when(fire_in)
                def _():
                    fire_input(c + NBUF, b)

            if wait_out:
                wait_output(b)
            fire_gather(b)

        def stage_back(c, b):
            wait_gather(b)
            pltpu.async_copy(row_v[b],
                             out_hbm.at[pl.ds((gbase + c * G) * IW, CHUNK)],
                             sem_o[b])

        stage_front(0, 0, fire_in=bool(NBUF < n_chunks), wait_out=False)
        for c in range(0, NBUF - 1):
            stage_back(c, c % NBUF)
            stage_front(c + 1, (c + 1) % NBUF,
                        fire_in=bool(c + 1 + NBUF < n_chunks),
                        wait_out=False)
        n_main = ((n_chunks - 1) // NBUF) * NBUF

        for c in range(NBUF - 1, NBUF):
            stage_back(c, c % NBUF)
            stage_front(c + 1, (c + 1) % NBUF,
                        fire_in=bool(c + 1 + NBUF < n_chunks),
                        wait_out=True)

        def outer(o, carry):
            for b in range(NBUF):
                c = o * NBUF + b
                stage_back(c, b)
                stage_front(c + 1, (b + 1) % NBUF,
                            fire_in=(c + 1 + NBUF < n_chunks),
                            wait_out=True)
            return carry

        lax.fori_loop(1, n_main // NBUF, outer, 0)
        for c in range(n_main, n_chunks - 1):
            stage_back(c, c % NBUF)
            stage_front(c + 1, (c + 1) % NBUF, fire_in=False, wait_out=True)
        stage_back(n_chunks - 1, (n_chunks - 1) % NBUF)

        for b in range(NBUF):
            wait_output(b)

    return sc_fn


def kernel(phoneme_ids, padding_mask, table, pos_bias):
    B, T = phoneme_ids.shape
    V, D = table.shape
    N = B * T
    VROWS = ((V + 1 + NS - 1) // NS) * NS

    ids = phoneme_ids.reshape(N // IW, IW).astype(jnp.int32)
    mask = padding_mask.reshape(N // IW, IW).astype(jnp.int32)
    idm = jnp.concatenate([ids, mask], axis=1)
    bias = pos_bias.reshape(D).astype(jnp.float32)
    table_p = jnp.concatenate(
        [table.astype(jnp.float32), jnp.zeros((VROWS - V, D), jnp.float32)],
        axis=0)

    sc_fn = _build_sc_call(N, V, D, VROWS)
    out = sc_fn(idm, table_p, bias)
    return out.reshape(B, T, D)


# speedup vs baseline: 1.0386x; 1.0269x over previous
import functools

import jax
import jax.numpy as jnp
from jax import lax
from jax.experimental import pallas as pl
from jax.experimental.pallas import tpu as pltpu
from jax.experimental.pallas import tpu_sc as plsc

NC, NS, L = 2, 16, 16
NW = NC * NS
IW = 128
G = 2
CHUNK = G * IW
NBUF = 3


def _build_sc_call(N, V, D, VROWS):
    n_per_w = N // NW
    n_chunks = n_per_w // CHUNK
    rows_per_tile = VROWS // NS
    mesh = plsc.VectorSubcoreMesh(
        core_axis_name="c", subcore_axis_name="s",
        num_cores=NC, num_subcores=NS)

    @functools.partial(
        pl.kernel,
        out_type=jax.ShapeDtypeStruct((N, D), jnp.float32),
        mesh=mesh,
        scratch_types=[
            pltpu.VMEM_SHARED((VROWS, D), jnp.float32),
            pltpu.VMEM((8, D), jnp.float32),
            pltpu.VMEM((D,), jnp.float32),
            [pltpu.VMEM((G, 2 * IW), jnp.int32) for _ in range(NBUF)],
            [pltpu.VMEM((G, IW), jnp.int32) for _ in range(NBUF)],
            [pltpu.VMEM((CHUNK, D), jnp.float32) for _ in range(NBUF)],
            [pltpu.SemaphoreType.DMA for _ in range(NBUF)],
            [pltpu.SemaphoreType.DMA for _ in range(NBUF)],
            [pltpu.SemaphoreType.DMA for _ in range(NBUF)],
        ],
    )
    def sc_fn(idm_hbm, table_hbm, bias_hbm, out_hbm,
              aug_sh, row1_v, bias_v, in_v, eff_v, row_v, sem_i, sem_o,
              sem_g):
        cid = lax.axis_index("c")
        sid = lax.axis_index("s")
        wid = sid * NC + cid

        gbase = wid * (n_per_w // IW)

        def fire_input(c, b):
            pltpu.async_copy(idm_hbm.at[pl.ds(gbase + c * G, G)],
                             in_v[b], sem_i[b])

        def wait_input(c, b):
            pltpu.make_async_copy(idm_hbm.at[pl.ds(gbase + c * G, G)],
                                  in_v[b], sem_i[b]).wait()

        def wait_output(b):
            pltpu.make_async_copy(row_v[b], out_hbm.at[pl.ds(0, CHUNK)],
                                  sem_o[b]).wait()

        for b in range(NBUF):
            fire_input(b, b)

        NTAB = VROWS // 8
        base = pl.multiple_of(sid * 8, 8)

        @pl.when(sid < NTAB)
        def _table_rows():
            bias_d = pltpu.async_copy(bias_hbm, bias_v, sem_g[0])
            rows_d = pltpu.async_copy(table_hbm.at[pl.ds(base, 8)], row1_v,
                                      sem_g[1])
            bias_d.wait()
            rows_d.wait()
            for k in range(8):
                @pl.when(base + k < V)
                def _add():
                    for j in range(D // L):
                        sl = pl.ds(j * L, L)
                        row1_v[k, sl] = row1_v[k, sl] + bias_v[sl]
            pltpu.sync_copy(row1_v, aug_sh.at[pl.ds(base, 8)])

        plsc.subcore_barrier()

        def fire_gather(b):
            for g in range(G):
                pltpu.async_copy(aug_sh.at[eff_v[b].at[g]],
                                 row_v[b].at[pl.ds(g * IW, IW)],
                                 sem_g[b])

        def wait_gather(b):
            for g in range(G):
                pltpu.make_async_copy(aug_sh.at[eff_v[b].at[g]],
                                      row_v[b].at[pl.ds(g * IW, IW)],
                                      sem_g[b]).wait()

        def stage_front(c, b, fire_in, wait_out):
            wait_input(c, b)

            def sel(k, c2):
                sl = pl.ds(k * L, L)
                msl = pl.ds(k * L + IW, L)
                for g in range(G):
                    idv = in_v[b][g, sl]
                    mv = in_v[b][g, msl]
                    eff_v[b][g, sl] = jnp.where(mv != 0, V, idv)
                return c2

            lax.fori_loop(0, IW // L, sel, 0)

            if isinstance(fire_in, bool):
                if fire_in:
                    fire_input(c + NBUF, b)
            else:
                @pl.when(fire_in)
                def _():
                    fire_input(c + NBUF, b)

            if wait_out:
                wait_output(b)
            fire_gather(b)

        def stage_back(c, b):
            wait_gather(b)
            pltpu.async_copy(row_v[b],
                             out_hbm.at[pl.ds((gbase + c * G) * IW, CHUNK)],
                             sem_o[b])

        stage_front(0, 0, fire_in=bool(NBUF < n_chunks), wait_out=False)
        for c in range(0, NBUF - 1):
            stage_back(c, c % NBUF)
            stage_front(c + 1, (c + 1) % NBUF,
                        fire_in=bool(c + 1 + NBUF < n_chunks),
                        wait_out=False)
        n_main = ((n_chunks - 1) // NBUF) * NBUF

        for c in range(NBUF - 1, NBUF):
            stage_back(c, c % NBUF)
            stage_front(c + 1, (c + 1) % NBUF,
                        fire_in=bool(c + 1 + NBUF < n_chunks),
                        wait_out=True)

        def outer(o, carry):
            for b in range(NBUF):
                c = o * NBUF + b
                stage_back(c, b)
                stage_front(c + 1, (b + 1) % NBUF,
                            fire_in=(c + 1 + NBUF < n_chunks),
                            wait_out=True)
            return carry

        lax.fori_loop(1, n_main // NBUF, outer, 0)
        for c in range(n_main, n_chunks - 1):
            stage_back(c, c % NBUF)
            stage_front(c + 1, (c + 1) % NBUF, fire_in=False, wait_out=True)
        stage_back(n_chunks - 1, (n_chunks - 1) % NBUF)

        for b in range(NBUF):
            wait_output(b)

    return sc_fn


def kernel(phoneme_ids, padding_mask, table, pos_bias):
    B, T = phoneme_ids.shape
    V, D = table.shape
    N = B * T
    VROWS = ((V + 1 + NS - 1) // NS) * NS

    ids = phoneme_ids.reshape(N // IW, IW).astype(jnp.int32)
    mask = padding_mask.reshape(N // IW, IW).astype(jnp.int32)
    idm = jnp.concatenate([ids, mask], axis=1)
    bias = pos_bias.reshape(D).astype(jnp.float32)
    table_p = jnp.concatenate(
        [table.astype(jnp.float32), jnp.zeros((VROWS - V, D), jnp.float32)],
        axis=0)

    sc_fn = _build_sc_call(N, V, D, VROWS)
    out = sc_fn(idm, table_p, bias)
    return out.reshape(B, T, D)
